# baseline (speedup 1.0000x reference)
import jax
import jax.numpy as jnp
from jax.experimental import pallas as pl
from jax.experimental.pallas import tpu as pltpu

_SIGMA = 2.0
_THR = 0.5
_N = 500
_HW = 256 * 256
_KC = 4096
_NK = _HW // _KC


def _gram_body(x_ref, g_ref, soft_ref):
    k = pl.program_id(0)

    @pl.when(k == 0)
    def _init():
        g_ref[...] = jnp.zeros_like(g_ref)
        soft_ref[...] = jnp.zeros_like(soft_ref)

    x = x_ref[...]
    m = x > _THR
    seg = jnp.where(m, 1.0, 0.0).astype(jnp.bfloat16)
    xs = jnp.where(m, x, 0.0)
    parts = [xs[:, i * 128:(i + 1) * 128] for i in range(_KC // 128)]
    while len(parts) > 1:
        nxt = [parts[i] + parts[i + 1] for i in range(0, len(parts) - 1, 2)]
        if len(parts) % 2:
            nxt.append(parts[-1])
        parts = nxt
    g_ref[...] += jax.lax.dot_general(
        seg, seg, (((1,), (1,)), ((), ())),
        preferred_element_type=jnp.float32)
    soft_ref[...] += parts[0]


def _nms_body(g_ref, soft_ref, cs_col_ref, lab_col_ref, lab_row_ref,
              upd_ref, sc_ref):
    g = g_ref[...]
    ia = jax.lax.broadcasted_iota(jnp.int32, (_N, _N), 0)
    ib = jax.lax.broadcasted_iota(jnp.int32, (_N, _N), 1)
    eye = ia == ib
    gd = jnp.where(eye, g, 0.0)
    sums_col = jnp.sum(gd, axis=1, keepdims=True)
    sums_row = jnp.sum(gd, axis=0, keepdims=True)
    soft_col = jnp.sum(soft_ref[...], axis=1, keepdims=True)
    seg_score = soft_col / jnp.maximum(sums_col, 1.0)
    scores_col = cs_col_ref[...] * seg_score
    scores_row = jnp.sum(jnp.where(eye, scores_col, 0.0), axis=0,
                         keepdims=True)

    union = sums_col + sums_row - g
    iou = g / jnp.maximum(union, 1.0)
    lmatch = lab_col_ref[...] == lab_row_ref[...]
    prec = (scores_col > scores_row) | ((scores_col == scores_row) & (ia < ib))
    succ = (scores_row > scores_col) | ((scores_row == scores_col) & (ib < ia))
    d = jnp.where(prec & lmatch, iou, 0.0)
    dt = jnp.where(succ & lmatch, iou, 0.0)
    c_col = jnp.max(dt, axis=1, keepdims=True)
    e = d * d - c_col * c_col
    m_row = jnp.max(e, axis=0, keepdims=True)
    coeff = jnp.exp(-_SIGMA * m_row)
    upd_ref[...] = scores_row * coeff
    sc_ref[...] = scores_row


def kernel(mask_preds, cate_scores, cate_labels):
    flat = mask_preds.reshape(_N, _HW)
    g, soft = pl.pallas_call(
        _gram_body,
        grid=(_NK,),
        in_specs=[pl.BlockSpec((_N, _KC), lambda k: (0, k))],
        out_specs=[pl.BlockSpec((_N, _N), lambda k: (0, 0)),
                   pl.BlockSpec((_N, 128), lambda k: (0, 0))],
        out_shape=[jax.ShapeDtypeStruct((_N, _N), jnp.float32),
                   jax.ShapeDtypeStruct((_N, 128), jnp.float32)],
        compiler_params=pltpu.CompilerParams(
            dimension_semantics=("arbitrary",),
            vmem_limit_bytes=48 * 1024 * 1024,
        ),
        name="solov2_gram",
    )(flat)

    upd, sc = pl.pallas_call(
        _nms_body,
        out_shape=[jax.ShapeDtypeStruct((1, _N), jnp.float32),
                   jax.ShapeDtypeStruct((1, _N), jnp.float32)],
        name="solov2_nms",
    )(g, soft,
      cate_scores.reshape(_N, 1),
      cate_labels.reshape(_N, 1),
      cate_labels.reshape(1, _N))

    order = jnp.argsort(-sc[0])
    return upd[0][order]


# speedup vs baseline: 1.2262x; 1.2262x over previous
import jax
import jax.numpy as jnp
from jax.experimental import pallas as pl
from jax.experimental.pallas import tpu as pltpu

_SIGMA = 2.0
_THR = 0.5
_N = 500
_H = 256
_W = 256


def _gram_body(x_hbm, g_ref, soft_ref, buf, sems):
    k = pl.program_id(0)

    @pl.when(k == 0)
    def _init():
        g_ref[...] = jnp.zeros_like(g_ref)
        soft_ref[...] = jnp.zeros_like(soft_ref)
        pltpu.make_async_copy(x_hbm.at[:, 0, :], buf.at[0], sems.at[0]).start()

    @pl.when(k < _H - 1)
    def _prefetch():
        nxt = k + 1
        pltpu.make_async_copy(
            x_hbm.at[:, nxt, :], buf.at[nxt % 2], sems.at[nxt % 2]).start()

    pltpu.make_async_copy(
        x_hbm.at[:, k, :], buf.at[k % 2], sems.at[k % 2]).wait()
    x = buf[k % 2]
    m = x > _THR
    seg = jnp.where(m, 1.0, 0.0).astype(jnp.bfloat16)
    xs = jnp.where(m, x, 0.0)
    g_ref[...] += jax.lax.dot_general(
        seg, seg, (((1,), (1,)), ((), ())),
        preferred_element_type=jnp.float32)
    soft_ref[...] += xs


def _nms_body(g_ref, soft_ref, cs_col_ref, lab_col_ref, lab_row_ref,
              upd_ref, sc_ref):
    g = g_ref[...]
    ia = jax.lax.broadcasted_iota(jnp.int32, (_N, _N), 0)
    ib = jax.lax.broadcasted_iota(jnp.int32, (_N, _N), 1)
    eye = ia == ib
    gd = jnp.where(eye, g, 0.0)
    sums_col = jnp.sum(gd, axis=1, keepdims=True)
    sums_row = jnp.sum(gd, axis=0, keepdims=True)
    soft_col = jnp.sum(soft_ref[...], axis=1, keepdims=True)

    seg_score = soft_col / jnp.maximum(sums_col, 1.0)
    scores_col = cs_col_ref[...] * seg_score
    scores_row = jnp.sum(jnp.where(eye, scores_col, 0.0), axis=0,
                         keepdims=True)

    union = sums_col + sums_row - g
    iou = g / jnp.maximum(union, 1.0)
    lmatch = lab_col_ref[...] == lab_row_ref[...]
    prec = (scores_col > scores_row) | ((scores_col == scores_row) & (ia < ib))
    succ = (scores_row > scores_col) | ((scores_row == scores_col) & (ib < ia))
    d = jnp.where(prec & lmatch, iou, 0.0)
    dt = jnp.where(succ & lmatch, iou, 0.0)
    c_col = jnp.max(dt, axis=1, keepdims=True)
    e = d * d - c_col * c_col
    m_row = jnp.max(e, axis=0, keepdims=True)
    coeff = jnp.exp(-_SIGMA * m_row)
    upd_ref[...] = scores_row * coeff
    sc_ref[...] = scores_row


def kernel(mask_preds, cate_scores, cate_labels):
    g, soft = pl.pallas_call(
        _gram_body,
        grid=(_H,),
        in_specs=[pl.BlockSpec(memory_space=pl.ANY)],
        out_specs=[pl.BlockSpec((_N, _N), lambda k: (0, 0)),
                   pl.BlockSpec((_N, _W), lambda k: (0, 0))],
        out_shape=[jax.ShapeDtypeStruct((_N, _N), jnp.float32),
                   jax.ShapeDtypeStruct((_N, _W), jnp.float32)],
        scratch_shapes=[pltpu.VMEM((2, _N, _W), jnp.float32),
                        pltpu.SemaphoreType.DMA((2,))],
        compiler_params=pltpu.CompilerParams(
            dimension_semantics=("arbitrary",),
            vmem_limit_bytes=48 * 1024 * 1024,
        ),
        name="solov2_gram",
    )(mask_preds)

    upd, sc = pl.pallas_call(
        _nms_body,
        out_shape=[jax.ShapeDtypeStruct((1, _N), jnp.float32),
                   jax.ShapeDtypeStruct((1, _N), jnp.float32)],
        name="solov2_nms",
    )(g, soft,
      cate_scores.reshape(_N, 1),
      cate_labels.reshape(_N, 1),
      cate_labels.reshape(1, _N))

    order = jnp.argsort(-sc[0])
    return upd[0][order]


# speedup vs baseline: 3.6879x; 3.0077x over previous
import jax
import jax.numpy as jnp
from jax.experimental import pallas as pl
from jax.experimental.pallas import tpu as pltpu

_SIGMA = 2.0
_THR = 0.5
_N = 500
_H = 256
_W = 256


_D = 8
_NP = 512


def _gram_body(x_hbm, g_ref, soft_ref, buf, sems):
    k = pl.program_id(0)

    @pl.when(k == 0)
    def _init():
        soft_ref[...] = jnp.zeros_like(soft_ref)
        for d in range(_D):
            pltpu.make_async_copy(
                x_hbm.at[:, d, :], buf.at[d], sems.at[d]).start()

    @pl.when((k >= 1) & (k <= _H - _D))
    def _prefetch():
        nxt = k + _D - 1
        pltpu.make_async_copy(
            x_hbm.at[:, nxt, :], buf.at[nxt % _D], sems.at[nxt % _D]).start()

    pltpu.make_async_copy(
        x_hbm.at[:, k, :], buf.at[k % _D], sems.at[k % _D]).wait()
    x = buf[k % _D]
    m = x > _THR
    seg = jnp.where(m, 1.0, 0.0).astype(jnp.float8_e4m3fn)
    seg = jnp.concatenate(
        [seg, jnp.zeros((_NP - _N, _W), jnp.float8_e4m3fn)], axis=0)
    xs = jnp.where(m, x, 0.0)
    for j in range(2):
        pltpu.matmul_push_rhs(seg[j * 256:(j + 1) * 256, :],
                              staging_register=0, mxu_index=j, transpose=True)
        pltpu.matmul_acc_lhs(0, seg, mxu_index=j, load_staged_rhs=0)
    soft_ref[...] += xs

    @pl.when(k == _H - 1)
    def _finish():
        g0 = pltpu.matmul_pop(0, (_NP, 256), jnp.float32, mxu_index=0)
        g1 = pltpu.matmul_pop(0, (_NP, 256), jnp.float32, mxu_index=1)
        g_ref[...] = jnp.concatenate([g0, g1], axis=1)[:_N, :_N]


def _nms_body(g_ref, soft_ref, cs_col_ref, lab_col_ref, lab_row_ref,
              upd_ref, sc_ref):
    g = g_ref[...]
    ia = jax.lax.broadcasted_iota(jnp.int32, (_N, _N), 0)
    ib = jax.lax.broadcasted_iota(jnp.int32, (_N, _N), 1)
    eye = ia == ib
    gd = jnp.where(eye, g, 0.0)
    sums_col = jnp.sum(gd, axis=1, keepdims=True)
    sums_row = jnp.sum(gd, axis=0, keepdims=True)
    soft_col = jnp.sum(soft_ref[...], axis=1, keepdims=True)

    seg_score = soft_col / jnp.maximum(sums_col, 1.0)
    scores_col = cs_col_ref[...] * seg_score
    scores_row = jnp.sum(jnp.where(eye, scores_col, 0.0), axis=0,
                         keepdims=True)

    union = sums_col + sums_row - g
    iou = g / jnp.maximum(union, 1.0)
    lmatch = lab_col_ref[...] == lab_row_ref[...]
    prec = (scores_col > scores_row) | ((scores_col == scores_row) & (ia < ib))
    succ = (scores_row > scores_col) | ((scores_row == scores_col) & (ib < ia))
    d = jnp.where(prec & lmatch, iou, 0.0)
    dt = jnp.where(succ & lmatch, iou, 0.0)
    c_col = jnp.max(dt, axis=1, keepdims=True)
    e = d * d - c_col * c_col
    m_row = jnp.max(e, axis=0, keepdims=True)
    coeff = jnp.exp(-_SIGMA * m_row)
    upd_ref[...] = scores_row * coeff
    sc_ref[...] = scores_row


def kernel(mask_preds, cate_scores, cate_labels):
    g, soft = pl.pallas_call(
        _gram_body,
        grid=(_H,),
        in_specs=[pl.BlockSpec(memory_space=pl.ANY)],
        out_specs=[pl.BlockSpec((_N, _N), lambda k: (0, 0)),
                   pl.BlockSpec((_N, _W), lambda k: (0, 0))],
        out_shape=[jax.ShapeDtypeStruct((_N, _N), jnp.float32),
                   jax.ShapeDtypeStruct((_N, _W), jnp.float32)],
        scratch_shapes=[pltpu.VMEM((_D, _N, _W), jnp.float32),
                        pltpu.SemaphoreType.DMA((_D,))],
        compiler_params=pltpu.CompilerParams(
            dimension_semantics=("arbitrary",),
            vmem_limit_bytes=48 * 1024 * 1024,
        ),
        name="solov2_gram",
    )(mask_preds)

    upd, sc = pl.pallas_call(
        _nms_body,
        out_shape=[jax.ShapeDtypeStruct((1, _N), jnp.float32),
                   jax.ShapeDtypeStruct((1, _N), jnp.float32)],
        name="solov2_nms",
    )(g, soft,
      cate_scores.reshape(_N, 1),
      cate_labels.reshape(_N, 1),
      cate_labels.reshape(1, _N))

    order = jnp.argsort(-sc[0])
    return upd[0][order]


# speedup vs baseline: 4.2564x; 1.1542x over previous
import jax
import jax.numpy as jnp
from jax.experimental import pallas as pl
from jax.experimental.pallas import tpu as pltpu

_SIGMA = 2.0
_THR = 0.5
_N = 500
_H = 256
_W = 256
_D = 8
_NP = 512


_R = 2


def _body(x_hbm, cs_col_ref, lab_col_ref, lab_row_ref, out_ref, buf, soft_ref,
          sems):
    k = pl.program_id(0)

    @pl.when(k == 0)
    def _init():
        soft_ref[...] = jnp.zeros_like(soft_ref)
        for d in range(_D):
            pltpu.make_async_copy(
                x_hbm.at[:, d, :], buf.at[d], sems.at[d]).start()

    @pl.when(k >= 1)
    def _prefetch():
        for r in range(_R):
            nxt = _R * k + _D - _R + r
            @pl.when(nxt < _H)
            def _():
                pltpu.make_async_copy(
                    x_hbm.at[:, nxt, :], buf.at[nxt % _D],
                    sems.at[nxt % _D]).start()

    acc = None
    for r in range(_R):
        h = _R * k + r
        pltpu.make_async_copy(
            x_hbm.at[:, h, :], buf.at[h % _D], sems.at[h % _D]).wait()
        x = buf[h % _D]
        m = x > _THR
        seg = jnp.where(m, 1.0, 0.0).astype(jnp.float8_e4m3fn)
        seg = jnp.concatenate(
            [seg, jnp.zeros((_NP - _N, _W), jnp.float8_e4m3fn)], axis=0)
        xs = jnp.where(m, x, 0.0)
        for j in range(2):
            pltpu.matmul_push_rhs(seg[j * 256:(j + 1) * 256, :],
                                  staging_register=0, mxu_index=j,
                                  transpose=True)
            pltpu.matmul_acc_lhs(0, seg, mxu_index=j, load_staged_rhs=0)
        acc = xs if acc is None else acc + xs
    soft_ref[...] += acc

    @pl.when(k == _H // _R - 1)
    def _finish():
        g0 = pltpu.matmul_pop(0, (_NP, 256), jnp.float32, mxu_index=0)
        g1 = pltpu.matmul_pop(0, (_NP, 256), jnp.float32, mxu_index=1)
        g = jnp.concatenate([g0, g1], axis=1)[:_N, :_N]

        ia = jax.lax.broadcasted_iota(jnp.int32, (_N, _N), 0)
        ib = jax.lax.broadcasted_iota(jnp.int32, (_N, _N), 1)
        eye = ia == ib
        gd = jnp.where(eye, g, 0.0)
        sums_col = jnp.sum(gd, axis=1, keepdims=True)
        sums_row = jnp.sum(gd, axis=0, keepdims=True)
        soft_col = jnp.sum(soft_ref[...], axis=1, keepdims=True)

        seg_score = soft_col / jnp.maximum(sums_col, 1.0)
        scores_col = cs_col_ref[...] * seg_score
        scores_row = jnp.sum(jnp.where(eye, scores_col, 0.0), axis=0,
                             keepdims=True)

        union = sums_col + sums_row - g
        iou = g / jnp.maximum(union, 1.0)
        lmatch = lab_col_ref[...] == lab_row_ref[...]
        prec = (scores_col > scores_row) | ((scores_col == scores_row)
                                            & (ia < ib))
        succ = (scores_row > scores_col) | ((scores_row == scores_col)
                                            & (ib < ia))
        d = jnp.where(prec & lmatch, iou, 0.0)
        dt = jnp.where(succ & lmatch, iou, 0.0)
        c_row = jnp.max(d, axis=0, keepdims=True)
        et = dt * dt - c_row * c_row
        coeff_col = jnp.exp(-_SIGMA * jnp.max(et, axis=1, keepdims=True))
        upd_col = scores_col * coeff_col
        rank_col = jnp.sum(jnp.where(succ, 1, 0), axis=1, keepdims=True)
        out_ref[...] = jnp.sum(jnp.where(rank_col == ib, upd_col, 0.0),
                               axis=0, keepdims=True)


def kernel(mask_preds, cate_scores, cate_labels):
    out = pl.pallas_call(
        _body,
        grid=(_H // _R,),
        in_specs=[pl.BlockSpec(memory_space=pl.ANY),
                  pl.BlockSpec((_N, 1), lambda k: (0, 0)),
                  pl.BlockSpec((_N, 1), lambda k: (0, 0)),
                  pl.BlockSpec((1, _N), lambda k: (0, 0))],
        out_specs=pl.BlockSpec((1, _N), lambda k: (0, 0)),
        out_shape=jax.ShapeDtypeStruct((1, _N), jnp.float32),
        scratch_shapes=[pltpu.VMEM((_D, _N, _W), jnp.float32),
                        pltpu.VMEM((_N, _W), jnp.float32),
                        pltpu.SemaphoreType.DMA((_D,))],
        compiler_params=pltpu.CompilerParams(
            dimension_semantics=("arbitrary",),
            vmem_limit_bytes=48 * 1024 * 1024,
        ),
        name="solov2_fused_nms",
    )(mask_preds,
      cate_scores.reshape(_N, 1),
      cate_labels.reshape(_N, 1),
      cate_labels.reshape(1, _N))
    return out[0]


# speedup vs baseline: 4.2728x; 1.0038x over previous
import jax
import jax.numpy as jnp
from jax.experimental import pallas as pl
from jax.experimental.pallas import tpu as pltpu

_SIGMA = 2.0
_THR = 0.5
_N = 500
_H = 256
_W = 256
_D = 12
_NP = 512


_R = 4


def _body(x_hbm, cs_col_ref, lab_col_ref, lab_row_ref, out_ref, buf, soft_ref,
          sems):
    k = pl.program_id(0)

    @pl.when(k == 0)
    def _init():
        soft_ref[...] = jnp.zeros_like(soft_ref)
        for d in range(_D):
            pltpu.make_async_copy(
                x_hbm.at[:, d, :], buf.at[d], sems.at[d]).start()

    @pl.when(k >= 1)
    def _prefetch():
        for r in range(_R):
            nxt = _R * k + _D - _R + r
            @pl.when(nxt < _H)
            def _():
                pltpu.make_async_copy(
                    x_hbm.at[:, nxt, :], buf.at[nxt % _D],
                    sems.at[nxt % _D]).start()

    acc = None
    for r in range(_R):
        h = _R * k + r
        pltpu.make_async_copy(
            x_hbm.at[:, h, :], buf.at[h % _D], sems.at[h % _D]).wait()
        x = buf[h % _D]
        m = x > _THR
        seg = jnp.where(m, 1.0, 0.0).astype(jnp.float8_e4m3fn)
        seg = jnp.concatenate(
            [seg, jnp.zeros((_NP - _N, _W), jnp.float8_e4m3fn)], axis=0)
        xs = jnp.where(m, x, 0.0)
        for j in range(2):
            pltpu.matmul_push_rhs(seg[j * 256:(j + 1) * 256, :],
                                  staging_register=0, mxu_index=j,
                                  transpose=True)
            pltpu.matmul_acc_lhs(0, seg, mxu_index=j, load_staged_rhs=0)
        acc = xs if acc is None else acc + xs
    soft_ref[...] += acc

    @pl.when(k == _H // _R - 1)
    def _finish():
        g0 = pltpu.matmul_pop(0, (_NP, 256), jnp.float32, mxu_index=0)
        g1 = pltpu.matmul_pop(0, (_NP, 256), jnp.float32, mxu_index=1)
        g = jnp.concatenate([g0, g1], axis=1)[:_N, :_N]

        ia = jax.lax.broadcasted_iota(jnp.int32, (_N, _N), 0)
        ib = jax.lax.broadcasted_iota(jnp.int32, (_N, _N), 1)
        eye = ia == ib
        gd = jnp.where(eye, g, 0.0)
        sums_col = jnp.sum(gd, axis=1, keepdims=True)
        sums_row = jnp.sum(gd, axis=0, keepdims=True)
        soft_col = jnp.sum(soft_ref[...], axis=1, keepdims=True)

        seg_score = soft_col / jnp.maximum(sums_col, 1.0)
        scores_col = cs_col_ref[...] * seg_score
        scores_row = jnp.sum(jnp.where(eye, scores_col, 0.0), axis=0,
                             keepdims=True)

        union = sums_col + sums_row - g
        iou = g / jnp.maximum(union, 1.0)
        lmatch = lab_col_ref[...] == lab_row_ref[...]
        prec = (scores_col > scores_row) | ((scores_col == scores_row)
                                            & (ia < ib))
        succ = (scores_row > scores_col) | ((scores_row == scores_col)
                                            & (ib < ia))
        d = jnp.where(prec & lmatch, iou, 0.0)
        dt = jnp.where(succ & lmatch, iou, 0.0)
        c_row = jnp.max(d, axis=0, keepdims=True)
        et = dt * dt - c_row * c_row
        coeff_col = jnp.exp(-_SIGMA * jnp.max(et, axis=1, keepdims=True))
        upd_col = scores_col * coeff_col
        rank_col = jnp.sum(jnp.where(succ, 1, 0), axis=1, keepdims=True)
        out_ref[...] = jnp.sum(jnp.where(rank_col == ib, upd_col, 0.0),
                               axis=0, keepdims=True)


def kernel(mask_preds, cate_scores, cate_labels):
    out = pl.pallas_call(
        _body,
        grid=(_H // _R,),
        in_specs=[pl.BlockSpec(memory_space=pl.ANY),
                  pl.BlockSpec((_N, 1), lambda k: (0, 0)),
                  pl.BlockSpec((_N, 1), lambda k: (0, 0)),
                  pl.BlockSpec((1, _N), lambda k: (0, 0))],
        out_specs=pl.BlockSpec((1, _N), lambda k: (0, 0)),
        out_shape=jax.ShapeDtypeStruct((1, _N), jnp.float32),
        scratch_shapes=[pltpu.VMEM((_D, _N, _W), jnp.float32),
                        pltpu.VMEM((_N, _W), jnp.float32),
                        pltpu.SemaphoreType.DMA((_D,))],
        compiler_params=pltpu.CompilerParams(
            dimension_semantics=("arbitrary",),
            vmem_limit_bytes=48 * 1024 * 1024,
        ),
        name="solov2_fused_nms",
    )(mask_preds,
      cate_scores.reshape(_N, 1),
      cate_labels.reshape(_N, 1),
      cate_labels.reshape(1, _N))
    return out[0]
